# baseline (speedup 1.0000x reference)
import jax
import jax.numpy as jnp
from jax.experimental import pallas as pl


def _vq_kernel(z_ref, dic_ref, d2_ref, out_ref):
    z = z_ref[...]
    dic = dic_ref[...]
    n = z.shape[0]
    k = dic.shape[0]
    cross = jax.lax.dot_general(
        z, dic, (((1,), (1,)), ((), ())),
        precision=jax.lax.Precision.DEFAULT,
        preferred_element_type=jnp.float32)
    z2 = jnp.sum(z * z, axis=1, keepdims=True)
    dist = (d2_ref[...] + z2) - 2.0 * cross
    minval = jnp.min(dist, axis=1, keepdims=True)
    iota = jax.lax.broadcasted_iota(jnp.int32, (n, k), 1).astype(jnp.float32)
    idx = jnp.min(jnp.where(dist == minval, iota, float(k)), axis=1,
                  keepdims=True)
    onehot = (iota == idx).astype(jnp.bfloat16)
    out_ref[...] = jax.lax.dot_general(
        onehot, dic, (((1,), (0,)), ((), ())),
        precision=jax.lax.Precision.DEFAULT,
        preferred_element_type=jnp.float32)


def kernel(ze, dictionary):
    b, t, d = ze.shape
    n = b * t
    k = dictionary.shape[0]
    z = ze.reshape(n, d)
    d2 = jnp.sum(dictionary**2, axis=1)[None, :]
    blk = 2048
    out = pl.pallas_call(
        _vq_kernel,
        grid=(n // blk,),
        in_specs=[
            pl.BlockSpec((blk, d), lambda i: (i, 0)),
            pl.BlockSpec((k, d), lambda i: (0, 0)),
            pl.BlockSpec((1, k), lambda i: (0, 0)),
        ],
        out_specs=pl.BlockSpec((blk, d), lambda i: (i, 0)),
        out_shape=jax.ShapeDtypeStruct((n, d), jnp.float32),
    )(z, dictionary, d2)
    return out.reshape(b, t, d)
